# baseline (speedup 1.0000x reference)
import functools
import math

import jax
import jax.numpy as jnp
from jax.experimental import pallas as pl
from jax.experimental.pallas import tpu as pltpu
from jax.experimental.pallas import tpu_sc as plsc

_HIDDEN = 64
_SCALE = math.sqrt(_HIDDEN)
_LANES = 16
_NW = 32
_KB = 2


def kernel(input_ids, table):
    batch, seq = input_ids.shape
    n = batch * seq
    idx = input_ids.reshape(n).astype(jnp.int32)
    bpw = batch // _NW
    nch = bpw // _KB
    c_idx = _KB * seq
    mesh = plsc.VectorSubcoreMesh(core_axis_name="c", subcore_axis_name="s")

    @functools.partial(
        pl.kernel,
        out_type=jax.ShapeDtypeStruct((batch, seq, _HIDDEN), table.dtype),
        mesh=mesh,
        compiler_params=pltpu.CompilerParams(use_tc_tiling_on_sc=False),
        scratch_types=[
            pltpu.VMEM((4, c_idx), jnp.int32),
            pltpu.VMEM((2, c_idx, _HIDDEN), jnp.float32),
            pltpu.SemaphoreType.DMA((4,)),
            pltpu.SemaphoreType.DMA((2,)),
            pltpu.SemaphoreType.DMA((2,)),
        ],
    )
    def gather_scale(tab_hbm, idx_hbm, out_hbm, idx_v, rows_v, isem, gsem, osem):
        wid = jax.lax.axis_index("s") * 2 + jax.lax.axis_index("c")
        b_base = wid * bpw

        def idx_dma(c):
            return pltpu.async_copy(
                idx_hbm.at[pl.ds((b_base + c * _KB) * seq, c_idx)],
                idx_v.at[c % 4],
                isem.at[c % 4],
            )

        def gather(c):
            return pltpu.async_copy(
                tab_hbm.at[idx_v.at[c % 4]], rows_v.at[c % 2], gsem.at[c % 2]
            )

        def out_dma(c, j):
            return pltpu.async_copy(
                rows_v.at[c % 2].at[pl.ds(j * seq, seq)],
                out_hbm.at[b_base + c * _KB + j],
                osem.at[c % 2],
            )

        def scale(c):
            rb = rows_v.at[c % 2]

            @pl.loop(0, c_idx)
            def _(r):
                for j in range(_HIDDEN // _LANES):
                    slc = (pl.ds(r, 1), pl.ds(j * _LANES, _LANES))
                    rb.at[*slc][...] = rb.at[*slc][...] * _SCALE

        idmas = {}
        for c in range(min(4, nch)):
            idmas[c] = idx_dma(c)
        idmas[0].wait()
        gathers = {0: gather(0)}
        odmas = {}
        for c in range(nch):
            if c + 1 < nch:
                idmas[c + 1].wait()
                if c >= 1:
                    for j in range(_KB):
                        odmas[(c - 1, j)].wait()
                gathers[c + 1] = gather(c + 1)
            gathers[c].wait()
            if c + 4 < nch:
                idmas[c + 4] = idx_dma(c + 4)
            scale(c)
            for j in range(_KB):
                odmas[(c, j)] = out_dma(c, j)
        for c in (nch - 1, nch - 2):
            if c >= 0:
                for j in range(_KB):
                    odmas[(c, j)].wait()

    return gather_scale(table, idx)


# speedup vs baseline: 1.2549x; 1.2549x over previous
import functools
import math

import jax
import jax.numpy as jnp
from jax.experimental import pallas as pl
from jax.experimental.pallas import tpu as pltpu
from jax.experimental.pallas import tpu_sc as plsc

_HIDDEN = 64
_SLOT = 128
_SCALE = math.sqrt(_HIDDEN)
_LANES = 16
_NW = 32
_C = 256


def kernel(input_ids, table):
    batch, seq = input_ids.shape
    n = batch * seq
    idx = input_ids.reshape(n).astype(jnp.int32)
    tab128 = jnp.pad(table, ((0, 0), (0, _SLOT - _HIDDEN)))
    npw = n // _NW
    nch = npw // _C
    mesh = plsc.VectorSubcoreMesh(core_axis_name="c", subcore_axis_name="s")

    @functools.partial(
        pl.kernel,
        out_type=jax.ShapeDtypeStruct((n, _HIDDEN), table.dtype),
        mesh=mesh,
        compiler_params=pltpu.CompilerParams(use_tc_tiling_on_sc=True),
        scratch_types=[
            pltpu.VMEM((npw,), jnp.int32),
            pltpu.VMEM((2, _C, _SLOT), jnp.float32),
            pltpu.VMEM((_C, _HIDDEN), jnp.float32),
            pltpu.SemaphoreType.DMA,
            pltpu.SemaphoreType.DMA((2,)),
            pltpu.SemaphoreType.DMA,
        ],
    )
    def gather_scale(
        tab_hbm, idx_hbm, out_hbm, idx_v, rows_v, packed_v, isem, gsem, osem
    ):
        wid = jax.lax.axis_index("s") * 2 + jax.lax.axis_index("c")
        base = wid * npw

        def gather(c):
            return pltpu.async_copy(
                tab_hbm.at[idx_v.at[pl.ds(c * _C, _C)]],
                rows_v.at[c % 2],
                gsem.at[c % 2],
            )

        def out_dma(c):
            return pltpu.async_copy(
                packed_v,
                out_hbm.at[pl.ds(base + c * _C, _C)],
                osem,
            )

        def pack_scale(c):
            rb = rows_v.at[c % 2]

            @pl.loop(0, _C)
            def _(r):
                for j in range(_HIDDEN // _LANES):
                    src = (pl.ds(r, 1), pl.ds(j * _LANES, _LANES))
                    packed_v.at[*src][...] = rb.at[*src][...] * _SCALE

        pltpu.async_copy(idx_hbm.at[pl.ds(base, npw)], idx_v, isem).wait()
        gathers = {0: gather(0)}
        odmas = {}
        for c in range(nch):
            if c + 1 < nch:
                gathers[c + 1] = gather(c + 1)
            gathers[c].wait()
            if c >= 1:
                odmas[c - 1].wait()
            pack_scale(c)
            odmas[c] = out_dma(c)
        odmas[nch - 1].wait()

    out = gather_scale(tab128, idx)
    return out.reshape(batch, seq, _HIDDEN)
